# Initial kernel scaffold; baseline (speedup 1.0000x reference)
import jax
import jax.numpy as jnp
from jax.experimental import pallas as pl


def kernel(feats, y_down):
    raise NotImplementedError("write your pallas kernel here")


# speedup vs baseline: 3.8401x; 3.8401x over previous
import jax
import jax.numpy as jnp
from jax.experimental import pallas as pl
from jax.experimental.pallas import tpu as pltpu

_NUM_CLASSES = 19
_HW_BLK = 2048


def _tc_body(feats_ref, lab_ref, sums_ref, nsum_ref, cnt_ref):
    b = pl.program_id(0)
    k = pl.program_id(1)

    @pl.when(jnp.logical_and(b == 0, k == 0))
    def _init():
        sums_ref[...] = jnp.zeros_like(sums_ref)
        nsum_ref[...] = jnp.zeros_like(nsum_ref)
        cnt_ref[...] = jnp.zeros_like(cnt_ref)

    f = feats_ref[0]
    lab = lab_ref[0, 0]
    classes = jax.lax.broadcasted_iota(jnp.int32, (_HW_BLK, _NUM_CLASSES), 1)
    onehot = (lab[:, None] == classes).astype(jnp.float32)

    sums_ref[...] += jnp.dot(f, onehot, preferred_element_type=jnp.float32)
    norms = jnp.sqrt(jnp.sum(f * f, axis=0))
    nsum_ref[...] += jnp.dot(norms[None, :], onehot,
                             preferred_element_type=jnp.float32)
    cnt_ref[...] += jnp.sum(onehot, axis=0, keepdims=True)


def kernel(feats, y_down):
    B, C, H, W = feats.shape
    HW = H * W
    f3 = feats.reshape(B, C, HW)
    lab3 = y_down.reshape(B, 1, HW)

    grid = (B, HW // _HW_BLK)
    sums, nsum, cnt = pl.pallas_call(
        _tc_body,
        grid=grid,
        in_specs=[
            pl.BlockSpec((1, C, _HW_BLK), lambda b, k: (b, 0, k)),
            pl.BlockSpec((1, 1, _HW_BLK), lambda b, k: (b, 0, k)),
        ],
        out_specs=[
            pl.BlockSpec((C, _NUM_CLASSES), lambda b, k: (0, 0)),
            pl.BlockSpec((1, _NUM_CLASSES), lambda b, k: (0, 0)),
            pl.BlockSpec((1, _NUM_CLASSES), lambda b, k: (0, 0)),
        ],
        out_shape=[
            jax.ShapeDtypeStruct((C, _NUM_CLASSES), jnp.float32),
            jax.ShapeDtypeStruct((1, _NUM_CLASSES), jnp.float32),
            jax.ShapeDtypeStruct((1, _NUM_CLASSES), jnp.float32),
        ],
    )(f3, lab3)

    counts = cnt[0]
    safe = jnp.maximum(counts, 1.0)
    b_c = (sums / safe[None, :]).T
    n_c = nsum[0] / safe
    return b_c, n_c
